# Initial kernel scaffold; baseline (speedup 1.0000x reference)
import jax
import jax.numpy as jnp
from jax.experimental import pallas as pl


def kernel(x, weight):
    raise NotImplementedError("write your pallas kernel here")


# speedup vs baseline: 1.8805x; 1.8805x over previous
import functools

import jax
import jax.numpy as jnp
from jax import lax
from jax.experimental import pallas as pl
from jax.experimental.pallas import tpu as pltpu
from jax.experimental.pallas import tpu_sc as plsc

N_EMBED = 1_000_000
D_MODEL = 64
B_TOTAL = 16384 * 50
NC, NS = 2, 16
NW = NC * NS
B_PER_W = B_TOTAL // NW
CHUNK = 128
NCHUNK = B_PER_W // CHUNK
NBUF = 4

_MESH = plsc.VectorSubcoreMesh(
    core_axis_name="c", subcore_axis_name="s", num_cores=NC, num_subcores=NS
)


@functools.partial(
    pl.kernel,
    out_type=jax.ShapeDtypeStruct((B_TOTAL, D_MODEL), jnp.float32),
    mesh=_MESH,
    scratch_types=[
        pltpu.VMEM((NCHUNK, CHUNK), jnp.int32),
        pltpu.VMEM((NBUF, CHUNK, D_MODEL), jnp.float32),
    ]
    + [pltpu.SemaphoreType.DMA] * NBUF,
    compiler_params=pltpu.CompilerParams(use_tc_tiling_on_sc=False),
)
def _embed_gather(idx_hbm, table_hbm, out_hbm, idx_v, rows_v, *sems):
    wid = lax.axis_index("s") * NC + lax.axis_index("c")
    base = wid * B_PER_W

    pltpu.sync_copy(idx_hbm.at[wid], idx_v)

    def fire(g, b):
        pltpu.async_copy(table_hbm.at[idx_v.at[g]], rows_v.at[b], sems[b])

    def drain(b):
        pltpu.make_async_copy(
            table_hbm.at[idx_v.at[0]], rows_v.at[b], sems[b]
        ).wait()

    def store(g, b):
        pltpu.sync_copy(rows_v.at[b], out_hbm.at[pl.ds(base + g * CHUNK, CHUNK)])

    for b in range(NBUF):
        fire(b, b)

    @pl.loop(0, NCHUNK - NBUF, step=NBUF)
    def _(t):
        for b in range(NBUF):
            g = t + b
            drain(b)
            store(g, b)
            fire(g + NBUF, b)

    for b in range(NBUF):
        drain(b)
        store(NCHUNK - NBUF + b, b)


def kernel(x, weight):
    idx = x.reshape(-1).astype(jnp.int32).reshape(NW, NCHUNK, CHUNK)
    out = _embed_gather(idx, weight)
    return out.reshape(x.shape + (D_MODEL,))
